# Initial kernel scaffold; baseline (speedup 1.0000x reference)
import jax
import jax.numpy as jnp
from jax.experimental import pallas as pl


def kernel(values, cg, sys_idx, i1, i2):
    raise NotImplementedError("write your pallas kernel here")


# speedup vs baseline: 9.6165x; 9.6165x over previous
import jax
import jax.numpy as jnp
from jax import lax
from jax.experimental import pallas as pl
from jax.experimental.pallas import tpu as pltpu
from jax.experimental.pallas import tpu_sc as plsc

NSYS = 4
NA = 48
NM = 5
NP = 64
ROW = NM * NP
ROWP = 384
NKEY = NSYS * NA * NA
STOT = 9216
NORB = 3456

KT = 512
ST = 512
NKT = NKEY // KT
NST = STOT // ST

NC = 2
NS = 16
NW = NC * NS
KPW = NKEY // NW
CHUNK = 96
WCH = KPW // CHUNK


def _seg_body(sys_r, i1_r, i2_r, v_r, acc_ref):
    kt = pl.program_id(0)
    st = pl.program_id(1)
    s_v = sys_r[0, 0]
    i1v = i1_r[0, 0]
    i2v = i2_r[0, 0]
    k1 = (s_v * NA + i1v) * NA + i2v
    col = lax.broadcasted_iota(jnp.int32, (ST, KT), 1) + kt * KT
    e1 = (k1[:, None] == col).astype(jnp.float32)
    d1 = lax.dot_general(e1, v_r[0], (((0,), (0,)), ((), ())),
                         preferred_element_type=jnp.float32)

    @pl.when(st == 0)
    def _init():
        acc_ref[0] = d1

    @pl.when(st != 0)
    def _accum():
        acc_ref[0] += d1


def _tc_segment_sum(values3, sys3, i13, i23):
    return pl.pallas_call(
        _seg_body,
        grid=(NKT, NST),
        in_specs=[
            pl.BlockSpec((1, 1, ST), lambda kt, st: (st, 0, 0)),
            pl.BlockSpec((1, 1, ST), lambda kt, st: (st, 0, 0)),
            pl.BlockSpec((1, 1, ST), lambda kt, st: (st, 0, 0)),
            pl.BlockSpec((1, ST, ROWP), lambda kt, st: (st, 0, 0)),
        ],
        out_specs=pl.BlockSpec((1, KT, ROWP), lambda kt, st: (kt, 0, 0)),
        out_shape=jax.ShapeDtypeStruct((NKT, KT, ROWP), jnp.float32),
    )(sys3, i13, i23, values3)


def _perm_body(acc_hbm, acct_out, vals_v, tidx_v):
    c = lax.axis_index("c")
    s = lax.axis_index("s")
    w = c * NS + s
    iota16 = lax.iota(jnp.int32, 16)
    for j in range(WCH):
        for t in range(CHUNK // 16):
            gi0, rem0 = divmod(t * 16, NA)
            g48 = w * (KPW // NA) + j * (CHUNK // NA) + gi0
            sys_j = g48 // NA
            a1_j = g48 - sys_j * NA
            base_v = jnp.full((16,), sys_j * (NA * NA) + a1_j, jnp.int32)
            tidx_v[j, pl.ds(t * 16, 16)] = base_v + (rem0 + iota16) * NA
        pltpu.sync_copy(acc_hbm.at[pl.ds(w * KPW + j * CHUNK, CHUNK)], vals_v)
        pltpu.sync_copy(vals_v, acct_out.at[tidx_v.at[j]])


def _sc_permute(acc2):
    run = pl.kernel(
        _perm_body,
        out_type=jax.ShapeDtypeStruct((NKEY, ROWP), jnp.float32),
        mesh=plsc.VectorSubcoreMesh(core_axis_name="c", subcore_axis_name="s",
                                    num_cores=NC, num_subcores=NS),
        scratch_types=[
            pltpu.VMEM((CHUNK, ROWP), jnp.float32),
            pltpu.VMEM((WCH, CHUNK), jnp.int32),
        ],
    )
    return run(acc2)


def _tc_body(accA, accB, cg_s, out_ref):
    A = accA[0, 0]
    B = accB[0, 0]

    d = []
    e = []
    for a in range(3):
        for b in range(3):
            d_ab = A[:, 0:NP] * cg_s[(a * 3 + b) * NM]
            e_ab = B[:, 0:NP] * cg_s[(b * 3 + a) * NM]
            for m in range(1, NM):
                d_ab = d_ab + A[:, m * NP:(m + 1) * NP] * cg_s[(a * 3 + b) * NM + m]
                e_ab = e_ab + B[:, m * NP:(m + 1) * NP] * cg_s[(b * 3 + a) * NM + m]
            d.append(d_ab)
            e.append(e_ab)

    stk1 = jnp.stack(d).reshape(3, 3, NA, 8, 8)
    stk2 = jnp.stack(e).reshape(3, 3, NA, 8, 8)
    t1 = stk1.transpose(3, 0, 2, 4, 1)
    t2 = stk2.transpose(4, 0, 2, 3, 1)
    blk_t = (0.5 * (t1 + t2)).reshape(24, NA, 24)

    padded = jnp.concatenate(
        [jnp.zeros((24, NA, 8), jnp.float32), blk_t,
         jnp.zeros((24, NA, 40), jnp.float32)], axis=2).reshape(24, NORB)
    out_ref[0, 0] = jnp.zeros((72, NORB), jnp.float32)
    out_ref[0, 0, 8:32, :] = padded


def _tc_build(acc6, acct6, cg_flat):
    return pl.pallas_call(
        _tc_body,
        grid=(NSYS, NA),
        in_specs=[
            pl.BlockSpec((1, 1, NA, ROWP), lambda s, a: (s, a, 0, 0)),
            pl.BlockSpec((1, 1, NA, ROWP), lambda s, a: (s, a, 0, 0)),
            pl.BlockSpec(memory_space=pltpu.SMEM),
        ],
        out_specs=pl.BlockSpec((1, 1, 72, NORB), lambda s, a: (s, a, 0, 0)),
        out_shape=jax.ShapeDtypeStruct((NSYS, NA, 72, NORB), jnp.float32),
    )(acc6, acct6, cg_flat)


def kernel(values, cg, sys_idx, i1, i2):
    values3 = jnp.pad(values.reshape(STOT, ROW),
                      ((0, 0), (0, ROWP - ROW))).reshape(NST, ST, ROWP)
    sys3 = sys_idx.astype(jnp.int32).reshape(NST, 1, ST)
    i13 = i1.astype(jnp.int32).reshape(NST, 1, ST)
    i23 = i2.astype(jnp.int32).reshape(NST, 1, ST)
    acc = _tc_segment_sum(values3, sys3, i13, i23).reshape(NKEY, ROWP)
    acct = _sc_permute(acc)
    acc6 = acc.reshape(NSYS, NA, NA, ROWP)
    acct6 = acct.reshape(NSYS, NA, NA, ROWP)
    h = _tc_build(acc6, acct6, cg.reshape(3 * 3 * NM))
    return h.reshape(NSYS, NORB, NORB)
